# Initial kernel scaffold; baseline (speedup 1.0000x reference)
import functools

import jax
import jax.numpy as jnp
from jax import lax
from jax.experimental import pallas as pl
from jax.experimental.pallas import tpu as pltpu
from jax.experimental.pallas import tpu_sc as plsc

NC, NS = 2, 16
NW = NC * NS
EB = 128
CW = 16


def _cdiv(a, b):
    return (a + b - 1) // b


def _make_gather(n_rows_pad, d, kx):
    mesh = plsc.VectorSubcoreMesh(core_axis_name="c", subcore_axis_name="s")

    @functools.partial(
        pl.kernel,
        out_type=jax.ShapeDtypeStruct((n_rows_pad, d), jnp.float32),
        mesh=mesh,
        scratch_types=[
            pltpu.VMEM((kx, EB), jnp.int32),
            pltpu.VMEM((EB, d), jnp.float32),
            pltpu.SemaphoreType.DMA,
        ],
    )
    def gather_kernel(table_hbm, idx_hbm, out_hbm, idx_v, rows_v, sem):
        cid = lax.axis_index("c")
        sid = lax.axis_index("s")
        wid = sid * NC + cid
        pltpu.sync_copy(idx_hbm.at[pl.ds(wid * kx, kx)], idx_v)

        @pl.loop(0, kx)
        def _(j):
            pltpu.async_copy(table_hbm.at[idx_v.at[j]], rows_v, sem).wait()
            pltpu.sync_copy(rows_v, out_hbm.at[pl.ds((wid * kx + j) * EB, EB)])

    return gather_kernel


def _make_edge_agg(n_nodes, n_acc, d, k_blocks, with_count):
    mesh = plsc.VectorSubcoreMesh(core_axis_name="c", subcore_axis_name="s")
    rows_out = n_nodes // NS
    rows_z = n_acc // NS

    out_type = [jax.ShapeDtypeStruct((NC, n_nodes, d), jnp.float32)]
    scratch = [
        pltpu.VMEM((k_blocks, EB), jnp.int32),
        pltpu.VMEM((k_blocks, EB), jnp.int32),
        pltpu.VMEM((EB, d), jnp.float32),
        pltpu.VMEM_SHARED((n_acc, d), jnp.float32),
        pltpu.SemaphoreType.DMA,
    ]
    if with_count:
        out_type.append(jax.ShapeDtypeStruct((NC, n_nodes, CW), jnp.float32))
        scratch += [
            pltpu.VMEM((EB, CW), jnp.float32),
            pltpu.VMEM_SHARED((n_acc, CW), jnp.float32),
        ]

    def body(g_hbm, src_hbm, dst_hbm, zg_hbm, zc_hbm, ones_hbm,
             *out_and_scratch):
        if with_count:
            (pg_hbm, pc_hbm, src_v, dst_v, rows_v, acc, sem,
             ones_v, acc_c) = out_and_scratch
        else:
            (pg_hbm, src_v, dst_v, rows_v, acc, sem) = out_and_scratch
        cid = lax.axis_index("c")
        sid = lax.axis_index("s")
        wid = cid * NS + sid

        pltpu.sync_copy(zg_hbm.at[pl.ds(sid * rows_z, rows_z)],
                        acc.at[pl.ds(sid * rows_z, rows_z)])
        if with_count:
            pltpu.sync_copy(zc_hbm.at[pl.ds(sid * rows_z, rows_z)],
                            acc_c.at[pl.ds(sid * rows_z, rows_z)])
            pltpu.sync_copy(ones_hbm, ones_v)
        pltpu.sync_copy(src_hbm.at[pl.ds(wid * k_blocks, k_blocks)], src_v)
        pltpu.sync_copy(dst_hbm.at[pl.ds(wid * k_blocks, k_blocks)], dst_v)
        plsc.subcore_barrier()

        @pl.loop(0, k_blocks)
        def _(j):
            pltpu.async_copy(g_hbm.at[src_v.at[j]], rows_v, sem).wait()
            pltpu.sync_copy(rows_v, acc.at[dst_v.at[j]], add=True)
            if with_count:
                pltpu.sync_copy(ones_v, acc_c.at[dst_v.at[j]], add=True)

        plsc.subcore_barrier()
        pltpu.sync_copy(acc.at[pl.ds(sid * rows_out, rows_out)],
                        pg_hbm.at[cid].at[pl.ds(sid * rows_out, rows_out)])
        if with_count:
            pltpu.sync_copy(acc_c.at[pl.ds(sid * rows_out, rows_out)],
                            pc_hbm.at[cid].at[pl.ds(sid * rows_out, rows_out)])

    return functools.partial(pl.kernel, out_type=tuple(out_type), mesh=mesh,
                             scratch_types=scratch)(body)


def _dot(a, b):
    return lax.dot_general(a, b, (((1,), (0,)), ((), ())),
                           precision=lax.Precision.HIGHEST,
                           preferred_element_type=jnp.float32)


def _mm2_body(h_ref, wl_ref, wr_ref, g_ref, u_ref):
    h = h_ref[...]
    g_ref[...] = _dot(h, wl_ref[...])
    u_ref[...] = _dot(h, wr_ref[...])


def _mid_body(pg_ref, pc_ref, u_ref, b_ref, wl_ref, wr_ref, g2_ref, u2_ref):
    pc = pc_ref[...]
    cnt = pc[0, :, :1] + pc[1, :, :1]
    inv = 1.0 / jnp.maximum(cnt, 1.0)
    mean = (pg_ref[0] + pg_ref[1]) * inv
    h1 = jnp.maximum(mean + b_ref[...] + u_ref[...], 0.0)
    g2_ref[...] = _dot(h1, wl_ref[...])
    u2_ref[...] = _dot(h1, wr_ref[...])


def _final_body(pg_ref, pc_ref, u_ref, b_ref, o_ref):
    pc = pc_ref[...]
    cnt = pc[0, :, :1] + pc[1, :, :1]
    inv = 1.0 / jnp.maximum(cnt, 1.0)
    mean = (pg_ref[0] + pg_ref[1]) * inv
    o_ref[...] = jnp.maximum(mean + b_ref[...] + u_ref[...], 0.0)


def kernel(x, edge_index, table, W1l, b1l, W1r, W2l, b2l, W2r):
    n = x.shape[0]
    e = edge_index.shape[1]
    d = table.shape[1]
    n_acc = n + CW

    k_blocks = _cdiv(e, NW * EB)
    e_pad = NW * k_blocks * EB
    src = edge_index[0]
    dst = edge_index[1]
    if e_pad != e:
        src = jnp.concatenate([src, jnp.zeros((e_pad - e,), jnp.int32)])
        dst = jnp.concatenate([dst, jnp.full((e_pad - e,), n, jnp.int32)])
    src2d = src.reshape(NW * k_blocks, EB)
    dst2d = dst.reshape(NW * k_blocks, EB)

    kx = _cdiv(n, NW * EB)
    n_pad = NW * kx * EB
    xp = jnp.concatenate([x, jnp.zeros((n_pad - n,), jnp.int32)]) if n_pad != n else x
    x2d = xp.reshape(NW * kx, EB)

    zeros_g = jnp.zeros((n_acc, d), jnp.float32)
    zeros_c = jnp.zeros((n_acc, CW), jnp.float32)
    ones_b = jnp.ones((EB, CW), jnp.float32)
    b1 = b1l.reshape(1, d)
    b2 = b2l.reshape(1, d)

    h0 = _make_gather(n_pad, d, kx)(table, x2d)

    bm = 1000
    grid = (n // bm,)
    g1, u1 = pl.pallas_call(
        _mm2_body,
        in_specs=[
            pl.BlockSpec((bm, d), lambda i: (i, 0)),
            pl.BlockSpec((d, d), lambda i: (0, 0)),
            pl.BlockSpec((d, d), lambda i: (0, 0)),
        ],
        out_specs=[
            pl.BlockSpec((bm, d), lambda i: (i, 0)),
            pl.BlockSpec((bm, d), lambda i: (i, 0)),
        ],
        out_shape=[
            jax.ShapeDtypeStruct((n, d), jnp.float32),
            jax.ShapeDtypeStruct((n, d), jnp.float32),
        ],
        grid=grid,
    )(h0, W1l, W1r)

    pg1, pc = _make_edge_agg(n, n_acc, d, k_blocks, True)(
        g1, src2d, dst2d, zeros_g, zeros_c, ones_b)

    g2, u2 = pl.pallas_call(
        _mid_body,
        in_specs=[
            pl.BlockSpec((NC, bm, d), lambda i: (0, i, 0)),
            pl.BlockSpec((NC, bm, CW), lambda i: (0, i, 0)),
            pl.BlockSpec((bm, d), lambda i: (i, 0)),
            pl.BlockSpec((1, d), lambda i: (0, 0)),
            pl.BlockSpec((d, d), lambda i: (0, 0)),
            pl.BlockSpec((d, d), lambda i: (0, 0)),
        ],
        out_specs=[
            pl.BlockSpec((bm, d), lambda i: (i, 0)),
            pl.BlockSpec((bm, d), lambda i: (i, 0)),
        ],
        out_shape=[
            jax.ShapeDtypeStruct((n, d), jnp.float32),
            jax.ShapeDtypeStruct((n, d), jnp.float32),
        ],
        grid=grid,
    )(pg1, pc, u1, b1, W2l, W2r)

    (pg2,) = _make_edge_agg(n, n_acc, d, k_blocks, False)(
        g2, src2d, dst2d, zeros_g, zeros_c, ones_b)

    out = pl.pallas_call(
        _final_body,
        in_specs=[
            pl.BlockSpec((NC, bm, d), lambda i: (0, i, 0)),
            pl.BlockSpec((NC, bm, CW), lambda i: (0, i, 0)),
            pl.BlockSpec((bm, d), lambda i: (i, 0)),
            pl.BlockSpec((1, d), lambda i: (0, 0)),
        ],
        out_specs=pl.BlockSpec((bm, d), lambda i: (i, 0)),
        out_shape=jax.ShapeDtypeStruct((n, d), jnp.float32),
        grid=grid,
    )(pg2, pc, u2, b2)
    return out


# speedup vs baseline: 3.0760x; 3.0760x over previous
import functools

import jax
import jax.numpy as jnp
from jax import lax
from jax.experimental import pallas as pl
from jax.experimental.pallas import tpu as pltpu
from jax.experimental.pallas import tpu_sc as plsc

NC, NS = 2, 16
NW = NC * NS
EB = 128


def _cdiv(a, b):
    return (a + b - 1) // b


def _make_gather(n_rows_pad, d):
    b_per_w = n_rows_pad // NW
    mesh = plsc.VectorSubcoreMesh(core_axis_name="c", subcore_axis_name="s")

    @functools.partial(
        pl.kernel,
        out_type=jax.ShapeDtypeStruct((n_rows_pad, d), jnp.float32),
        mesh=mesh,
        scratch_types=[
            pltpu.VMEM((b_per_w,), jnp.int32),
            pltpu.VMEM((b_per_w, d), jnp.float32),
            pltpu.SemaphoreType.DMA,
        ],
    )
    def gather_kernel(table_hbm, idx_hbm, out_hbm, idx_v, rows_v, sem):
        wid = lax.axis_index("s") * NC + lax.axis_index("c")
        base = wid * b_per_w
        pltpu.sync_copy(idx_hbm.at[pl.ds(base, b_per_w)], idx_v)
        pltpu.async_copy(table_hbm.at[idx_v], rows_v, sem).wait()
        pltpu.sync_copy(rows_v, out_hbm.at[pl.ds(base, b_per_w)])

    return gather_kernel


def _make_edge_agg(n_acc, d, k_blocks):
    mesh = plsc.VectorSubcoreMesh(core_axis_name="c", subcore_axis_name="s")
    rows_z = n_acc // NS

    @functools.partial(
        pl.kernel,
        out_type=jax.ShapeDtypeStruct((NC, n_acc, d), jnp.float32),
        mesh=mesh,
        scratch_types=[
            pltpu.VMEM((EB,), jnp.int32),
            pltpu.VMEM((EB,), jnp.int32),
            pltpu.VMEM((EB, d), jnp.float32),
            pltpu.VMEM_SHARED((n_acc, d), jnp.float32),
            pltpu.SemaphoreType.DMA,
        ],
    )
    def edge_agg(g_hbm, src_hbm, dst_hbm, zg_hbm, pg_hbm,
                 src_blk, dst_blk, rows_v, acc, sem):
        cid = lax.axis_index("c")
        sid = lax.axis_index("s")
        wid = sid * NC + cid

        pltpu.sync_copy(zg_hbm.at[pl.ds(sid * rows_z, rows_z)],
                        acc.at[pl.ds(sid * rows_z, rows_z)])
        plsc.subcore_barrier()

        @pl.loop(0, k_blocks)
        def _(j):
            base = (wid * k_blocks + j) * EB
            pltpu.sync_copy(src_hbm.at[pl.ds(base, EB)], src_blk)
            pltpu.sync_copy(dst_hbm.at[pl.ds(base, EB)], dst_blk)
            pltpu.async_copy(g_hbm.at[src_blk], rows_v, sem).wait()
            pltpu.sync_copy(rows_v, acc.at[dst_blk], add=True)

        plsc.subcore_barrier()
        pltpu.sync_copy(acc.at[pl.ds(sid * rows_z, rows_z)],
                        pg_hbm.at[cid].at[pl.ds(sid * rows_z, rows_z)])

    return edge_agg


def kernel(x, edge_index, table, W1l, b1l, W1r, W2l, b2l, W2r):
    n = x.shape[0]
    e = edge_index.shape[1]
    d = table.shape[1]
    n_acc = _cdiv(n + 1, 128) * 128

    n_pad = _cdiv(n, 8 * NW) * 8 * NW
    xp = jnp.concatenate([x, jnp.zeros((n_pad - n,), jnp.int32)]) if n_pad != n else x
    h0 = _make_gather(n_pad, d)(table, xp)[:n]

    k_blocks = _cdiv(e, NW * EB)
    e_pad = NW * k_blocks * EB
    src = edge_index[0]
    dst = edge_index[1]
    if e_pad != e:
        src = jnp.concatenate([src, jnp.zeros((e_pad - e,), jnp.int32)])
        dst = jnp.concatenate([dst, jnp.full((e_pad - e,), n, jnp.int32)])

    zeros_g = jnp.zeros((n_acc, d), jnp.float32)
    cnt = jax.ops.segment_sum(jnp.ones((e,), jnp.float32), edge_index[1],
                              num_segments=n)
    inv = 1.0 / jnp.maximum(cnt, 1.0)[:, None]

    g1 = h0 @ W1l
    pg = _make_edge_agg(n_acc, d, k_blocks)(g1, src, dst, zeros_g)
    mean = (pg[0] + pg[1])[:n] * inv
    h = jax.nn.relu(mean + b1l + h0 @ W1r)

    g2 = h @ W2l
    pg2 = _make_edge_agg(n_acc, d, k_blocks)(g2, src, dst, zeros_g)
    mean2 = (pg2[0] + pg2[1])[:n] * inv
    h2 = jax.nn.relu(mean2 + b2l + h @ W2r)
    return h2


# speedup vs baseline: 3.7344x; 1.2140x over previous
import functools

import jax
import jax.numpy as jnp
from jax import lax
from jax.experimental import pallas as pl
from jax.experimental.pallas import tpu as pltpu
from jax.experimental.pallas import tpu_sc as plsc

NC, NS = 2, 16
NW = NC * NS
EB = 128
VL = 16


def _cdiv(a, b):
    return (a + b - 1) // b


def _make_gather(n_rows_pad, d):
    b_per_w = n_rows_pad // NW
    mesh = plsc.VectorSubcoreMesh(core_axis_name="c", subcore_axis_name="s")

    @functools.partial(
        pl.kernel,
        out_type=jax.ShapeDtypeStruct((n_rows_pad, d), jnp.float32),
        mesh=mesh,
        scratch_types=[
            pltpu.VMEM((b_per_w,), jnp.int32),
            pltpu.VMEM((b_per_w, d), jnp.float32),
            pltpu.SemaphoreType.DMA,
        ],
    )
    def gather_kernel(table_hbm, idx_hbm, out_hbm, idx_v, rows_v, sem):
        wid = lax.axis_index("s") * NC + lax.axis_index("c")
        base = wid * b_per_w
        pltpu.sync_copy(idx_hbm.at[pl.ds(base, b_per_w)], idx_v)
        pltpu.async_copy(table_hbm.at[idx_v], rows_v, sem).wait()
        pltpu.sync_copy(rows_v, out_hbm.at[pl.ds(base, b_per_w)])

    return gather_kernel


def _make_edge_agg(n_acc, d, k_blocks):
    mesh = plsc.VectorSubcoreMesh(core_axis_name="c", subcore_axis_name="s")
    rows_z = n_acc // NS

    @functools.partial(
        pl.kernel,
        out_type=jax.ShapeDtypeStruct((NC, n_acc, d), jnp.float32),
        mesh=mesh,
        scratch_types=[
            pltpu.VMEM((EB,), jnp.int32),
            pltpu.VMEM((EB,), jnp.int32),
            pltpu.VMEM((EB, d), jnp.float32),
            pltpu.VMEM_SHARED((n_acc, d), jnp.float32),
            pltpu.SemaphoreType.DMA,
        ],
    )
    def edge_agg(g_hbm, src_hbm, dst_hbm, zg_hbm, pg_hbm,
                 src_blk, dst_blk, rows_v, acc, sem):
        cid = lax.axis_index("c")
        sid = lax.axis_index("s")
        wid = sid * NC + cid

        pltpu.sync_copy(zg_hbm.at[pl.ds(sid * rows_z, rows_z)],
                        acc.at[pl.ds(sid * rows_z, rows_z)])
        plsc.subcore_barrier()

        @pl.loop(0, k_blocks)
        def _(j):
            base = (wid * k_blocks + j) * EB
            pltpu.sync_copy(src_hbm.at[pl.ds(base, EB)], src_blk)
            pltpu.sync_copy(dst_hbm.at[pl.ds(base, EB)], dst_blk)
            pltpu.async_copy(g_hbm.at[src_blk], rows_v, sem).wait()
            pltpu.sync_copy(rows_v, acc.at[dst_blk], add=True)

        plsc.subcore_barrier()
        pltpu.sync_copy(acc.at[pl.ds(sid * rows_z, rows_z)],
                        pg_hbm.at[cid].at[pl.ds(sid * rows_z, rows_z)])

    return edge_agg


def _make_counts(n_acc, d, k_blocks):
    mesh = plsc.VectorSubcoreMesh(core_axis_name="c", subcore_axis_name="s")
    rows_z = n_acc // NS

    @functools.partial(
        pl.kernel,
        out_type=jax.ShapeDtypeStruct((NC, n_acc, d), jnp.float32),
        mesh=mesh,
        scratch_types=[
            pltpu.VMEM((EB,), jnp.int32),
            pltpu.VMEM((EB, d), jnp.float32),
            pltpu.VMEM_SHARED((n_acc, d), jnp.float32),
        ],
    )
    def counts_kernel(dst_hbm, zc_hbm, ones_hbm, out_hbm,
                      dst_blk, ones_v, acc):
        cid = lax.axis_index("c")
        sid = lax.axis_index("s")
        wid = sid * NC + cid

        pltpu.sync_copy(zc_hbm.at[pl.ds(sid * rows_z, rows_z)],
                        acc.at[pl.ds(sid * rows_z, rows_z)])
        pltpu.sync_copy(ones_hbm, ones_v)
        plsc.subcore_barrier()

        @pl.loop(0, k_blocks)
        def _(j):
            base = (wid * k_blocks + j) * EB
            pltpu.sync_copy(dst_hbm.at[pl.ds(base, EB)], dst_blk)
            pltpu.sync_copy(ones_v, acc.at[dst_blk], add=True)

        plsc.subcore_barrier()
        pltpu.sync_copy(acc.at[pl.ds(sid * rows_z, rows_z)],
                        out_hbm.at[cid].at[pl.ds(sid * rows_z, rows_z)])

    return counts_kernel


def _make_combine(n_rows, d, rb):
    grid = n_rows // rb

    def combine_body(pg_ref, pc_ref, h_ref, wl_ref, b_ref, wr_ref, out_ref):
        agg = pg_ref[0] + pg_ref[1]
        cnt = pc_ref[0, :, 0:1] + pc_ref[1, :, 0:1]
        inv = 1.0 / jnp.maximum(cnt, 1.0)
        mean = agg * inv
        out = (jnp.dot(mean, wl_ref[...], preferred_element_type=jnp.float32)
               + b_ref[...]
               + jnp.dot(h_ref[...], wr_ref[...],
                         preferred_element_type=jnp.float32))
        out_ref[...] = jnp.maximum(out, 0.0)

    return pl.pallas_call(
        combine_body,
        grid=(grid,),
        in_specs=[
            pl.BlockSpec((NC, rb, d), lambda i: (0, i, 0)),
            pl.BlockSpec((NC, rb, d), lambda i: (0, i, 0)),
            pl.BlockSpec((rb, d), lambda i: (i, 0)),
            pl.BlockSpec((d, d), lambda i: (0, 0)),
            pl.BlockSpec((1, d), lambda i: (0, 0)),
            pl.BlockSpec((d, d), lambda i: (0, 0)),
        ],
        out_specs=pl.BlockSpec((rb, d), lambda i: (i, 0)),
        out_shape=jax.ShapeDtypeStruct((n_rows, d), jnp.float32),
    )


def kernel(x, edge_index, table, W1l, b1l, W1r, W2l, b2l, W2r):
    n = x.shape[0]
    e = edge_index.shape[1]
    d = table.shape[1]

    rb = 512
    P = _cdiv(n + 1, 2560) * 2560
    xp = jnp.concatenate([x, jnp.zeros((P - n,), jnp.int32)])
    h0 = _make_gather(P, d)(table, xp)

    k_blocks = _cdiv(e, NW * EB)
    e_pad = NW * k_blocks * EB
    src = edge_index[0]
    dst = edge_index[1]
    if e_pad != e:
        src = jnp.concatenate([src, jnp.zeros((e_pad - e,), jnp.int32)])
        dst = jnp.concatenate([dst, jnp.full((e_pad - e,), n, jnp.int32)])

    zeros_g = jnp.zeros((P, d), jnp.float32)
    zeros_c = jnp.zeros((P, d), jnp.float32)

    edge_agg = _make_edge_agg(P, d, k_blocks)
    combine = _make_combine(P, d, rb)

    b1 = b1l.reshape(1, d)
    b2 = b2l.reshape(1, d)

    ones_e = jnp.ones((EB, d), jnp.float32)
    pc = _make_counts(P, d, k_blocks)(dst, zeros_c, ones_e)
    pg1 = edge_agg(h0, src, dst, zeros_g)
    h1 = combine(pg1, pc, h0, W1l, b1, W1r)
    pg2 = edge_agg(h1, src, dst, zeros_g)
    h2 = combine(pg2, pc, h1, W2l, b2, W2r)
    return h2[:n]
